# Initial kernel scaffold; baseline (speedup 1.0000x reference)
import jax
import jax.numpy as jnp
import numpy as np
from jax import lax
from jax.experimental import pallas as pl
from jax.experimental.pallas import tpu as pltpu
from jax.experimental.pallas import tpu_sc as plsc

N = 10000
E = 320000
D = 128
H = 4
DH = 32
INV_SQRT_DH = np.float32(1.0 / np.sqrt(DH))

NC = 2
NS = 16
EDGES_PER_TILE = E // NS
CHUNK = 80
N_CHUNKS = EDGES_PER_TILE // CHUNK
ROWS_PER_TILE = N // NS
ZROWS = 125
AW = 80


def _node_qkv_body(x_ref, wq_ref, wk_ref, wv_ref, g_ref, b_ref, qt_ref, kt_ref,
                   vt_ref):
    x = x_ref[...]
    m = jnp.mean(x, axis=0, keepdims=True)
    v = jnp.mean((x - m) ** 2, axis=0, keepdims=True)
    xn = (x - m) * lax.rsqrt(v + 1e-5) * g_ref[0:1, :] + b_ref[0:1, :]
    q = jnp.dot(xn, wq_ref[...].T, preferred_element_type=jnp.float32)
    k = jnp.dot(xn, wk_ref[...].T, preferred_element_type=jnp.float32)
    k = k * INV_SQRT_DH
    w = jnp.dot(xn, wv_ref[...].T, preferred_element_type=jnp.float32)
    qt_ref[0] = q[:, 0:64]
    qt_ref[1] = q[:, 64:128]
    kt_ref[0] = k[:, 0:64]
    kt_ref[1] = k[:, 64:128]
    vt_ref[0] = w[:, 0:64]
    vt_ref[1] = w[:, 64:128]


def _node_qkv(x, wq, wk, wv, g, b):
    g8 = jnp.broadcast_to(g[None, :], (8, D))
    b8 = jnp.broadcast_to(b[None, :], (8, D))
    out = jax.ShapeDtypeStruct((NC, N, 64), jnp.float32)
    return pl.pallas_call(
        _node_qkv_body,
        out_shape=(out, out, out),
    )(x, wq, wk, wv, g8, b8)


EBLK = 2500
N_EBLK = E // EBLK


def _edge_stats_body(e_ref, s_ref, q_ref):
    i = pl.program_id(0)
    blk = e_ref[...]
    ps = jnp.sum(blk, axis=0, keepdims=True)
    pq = jnp.sum(blk * blk, axis=0, keepdims=True)
    ps8 = jnp.broadcast_to(ps, (8, D))
    pq8 = jnp.broadcast_to(pq, (8, D))

    @pl.when(i == 0)
    def _():
        s_ref[...] = ps8
        q_ref[...] = pq8

    @pl.when(i > 0)
    def _():
        s_ref[...] += ps8
        q_ref[...] += pq8


def _edge_stats(e):
    return pl.pallas_call(
        _edge_stats_body,
        grid=(N_EBLK,),
        in_specs=[pl.BlockSpec((EBLK, D), lambda i: (i, 0))],
        out_specs=(pl.BlockSpec((8, D), lambda i: (0, 0)),
                   pl.BlockSpec((8, D), lambda i: (0, 0))),
        out_shape=(jax.ShapeDtypeStruct((8, D), jnp.float32),
                   jax.ShapeDtypeStruct((8, D), jnp.float32)),
    )(e)


def _proj_e_body(e_ref, w_ref, b_ref, o_ref):
    p = jnp.dot(e_ref[...], w_ref[...].T, preferred_element_type=jnp.float32)
    p = p + b_ref[0:1, :]
    o_ref[0] = p[:, 0:64]
    o_ref[1] = p[:, 64:128]


def _proj_e(e, w_eff, b_eff):
    b8 = jnp.broadcast_to(b_eff[None, :], (8, D))
    return pl.pallas_call(
        _proj_e_body,
        grid=(N_EBLK,),
        in_specs=[pl.BlockSpec((EBLK, D), lambda i: (i, 0)),
                  pl.BlockSpec((D, D), lambda i: (0, 0)),
                  pl.BlockSpec((8, D), lambda i: (0, 0))],
        out_specs=pl.BlockSpec((NC, EBLK, 64), lambda i: (0, i, 0)),
        out_shape=jax.ShapeDtypeStruct((NC, E, 64), jnp.float32),
    )(e, w_eff, b8)


def _sc_edge_body(qt, kt, vt, pe, src, dst, out,
                  src_v, dst_v, sidx_v, didx_v,
                  krows, qrows, vrows, pe_v, contrib, zbuf, acc, sem):
    c = lax.axis_index("c")
    s = lax.axis_index("s")

    def _zero_row(i, _):
        for j in range(AW // 16):
            zbuf[i, pl.ds(16 * j, 16)] = jnp.zeros((16,), jnp.float32)
        return 0

    lax.fori_loop(0, ZROWS, _zero_row, 0)
    for k in range(ROWS_PER_TILE // ZROWS):
        pltpu.sync_copy(zbuf, acc.at[pl.ds(s * ROWS_PER_TILE + k * ZROWS,
                                           ZROWS)])
    plsc.subcore_barrier()

    def _chunk(i, _):
        base = s * EDGES_PER_TILE + i * CHUNK
        pltpu.sync_copy(src.at[pl.ds(base, CHUNK)], src_v)
        pltpu.sync_copy(dst.at[pl.ds(base, CHUNK)], dst_v)
        off = c * N
        for j in range(CHUNK // 16):
            sl = pl.ds(16 * j, 16)
            sidx_v[sl] = src_v[sl] + off
            didx_v[sl] = dst_v[sl] + off
        g1 = pltpu.async_copy(kt.at[sidx_v], krows, sem)
        g2 = pltpu.async_copy(vt.at[sidx_v], vrows, sem)
        g3 = pltpu.async_copy(qt.at[didx_v], qrows, sem)
        g4 = pltpu.async_copy(pe.at[pl.ds(c * E + base, CHUNK)], pe_v, sem)
        g1.wait()
        g2.wait()
        g3.wait()
        g4.wait()

        lanes = lax.iota(jnp.int32, 16)

        def _edge(e, _):
            k0 = krows[e, pl.ds(0, 16)]
            k1 = krows[e, pl.ds(16, 16)]
            k2 = krows[e, pl.ds(32, 16)]
            k3 = krows[e, pl.ds(48, 16)]
            q0 = qrows[e, pl.ds(0, 16)]
            q1 = qrows[e, pl.ds(16, 16)]
            q2 = qrows[e, pl.ds(32, 16)]
            q3 = qrows[e, pl.ds(48, 16)]
            p0 = pe_v[e, pl.ds(0, 16)]
            p1 = pe_v[e, pl.ds(16, 16)]
            p2 = pe_v[e, pl.ds(32, 16)]
            p3 = pe_v[e, pl.ds(48, 16)]
            t0 = jnp.clip(k0 * q0, -5.0, 5.0) * p0
            t1 = jnp.clip(k1 * q1, -5.0, 5.0) * p1
            t2 = jnp.clip(k2 * q2, -5.0, 5.0) * p2
            t3 = jnp.clip(k3 * q3, -5.0, 5.0) * p3
            s0 = jnp.sum(t0 + t1, axis=0)
            s1 = jnp.sum(t2 + t3, axis=0)
            w0 = jnp.exp(jnp.clip(jnp.broadcast_to(s0, (16,)), -5.0, 5.0))
            w1 = jnp.exp(jnp.clip(jnp.broadcast_to(s1, (16,)), -5.0, 5.0))
            contrib[e, pl.ds(0, 16)] = vrows[e, pl.ds(0, 16)] * w0
            contrib[e, pl.ds(16, 16)] = vrows[e, pl.ds(16, 16)] * w0
            contrib[e, pl.ds(32, 16)] = vrows[e, pl.ds(32, 16)] * w1
            contrib[e, pl.ds(48, 16)] = vrows[e, pl.ds(48, 16)] * w1
            zv = jnp.where(lanes == 0, w0, jnp.where(lanes == 1, w1, 0.0))
            contrib[e, pl.ds(64, 16)] = zv
            return 0

        lax.fori_loop(0, CHUNK, _edge, 0)
        pltpu.sync_copy(contrib, acc.at[dst_v], add=True)
        return 0

    lax.fori_loop(0, N_CHUNKS, _chunk, 0)
    plsc.subcore_barrier()

    for k in range(ROWS_PER_TILE // ZROWS):
        r0 = s * ROWS_PER_TILE + k * ZROWS
        pltpu.sync_copy(acc.at[pl.ds(r0, ZROWS)], zbuf)
        pltpu.sync_copy(zbuf, out.at[c].at[pl.ds(r0, ZROWS)])


def _sc_edge(qt, kt, vt, pe, src, dst):
    qt2 = qt.reshape(NC * N, 64)
    kt2 = kt.reshape(NC * N, 64)
    vt2 = vt.reshape(NC * N, 64)
    pe2 = pe.reshape(NC * E, 64)
    mesh = plsc.VectorSubcoreMesh(core_axis_name="c", subcore_axis_name="s",
                                  num_cores=NC, num_subcores=NS)
    f = pl.kernel(
        _sc_edge_body,
        out_type=jax.ShapeDtypeStruct((NC, N, AW), jnp.float32),
        mesh=mesh,
        scratch_types=[
            pltpu.VMEM((CHUNK,), jnp.int32),
            pltpu.VMEM((CHUNK,), jnp.int32),
            pltpu.VMEM((CHUNK,), jnp.int32),
            pltpu.VMEM((CHUNK,), jnp.int32),
            pltpu.VMEM((CHUNK, 64), jnp.float32),
            pltpu.VMEM((CHUNK, 64), jnp.float32),
            pltpu.VMEM((CHUNK, 64), jnp.float32),
            pltpu.VMEM((CHUNK, 64), jnp.float32),
            pltpu.VMEM((CHUNK, AW), jnp.float32),
            pltpu.VMEM((ZROWS, AW), jnp.float32),
            pltpu.VMEM_SHARED((N, AW), jnp.float32),
            pltpu.SemaphoreType.DMA,
        ],
    )
    return f(qt2, kt2, vt2, pe2, src, dst)


def _final_body(acc_ref, x1_ref, wo_ref, bo_ref, w1_ref, w2_ref, g2_ref,
                b2_ref, wih_ref, whh_ref, bih_ref, bhh_ref, out_ref):
    a0 = acc_ref[0]
    a1 = acc_ref[1]
    wv = jnp.concatenate([a0[:, 0:64], a1[:, 0:64]], axis=1)
    z0 = a0[:, 64:65]
    z1 = a0[:, 65:66]
    z2 = a1[:, 64:65]
    z3 = a1[:, 65:66]
    den = jnp.concatenate([
        jnp.broadcast_to(z0, (N, DH)),
        jnp.broadcast_to(z1, (N, DH)),
        jnp.broadcast_to(z2, (N, DH)),
        jnp.broadcast_to(z3, (N, DH)),
    ], axis=1) + 1e-6
    h = wv / den
    h = jnp.dot(h, wo_ref[...].T, preferred_element_type=jnp.float32)
    h = h + bo_ref[0:1, :]
    x = x1_ref[...] + h
    x_in2 = x
    m = jnp.mean(x, axis=0, keepdims=True)
    v = jnp.mean((x - m) ** 2, axis=0, keepdims=True)
    xn = (x - m) * lax.rsqrt(v + 1e-5) * g2_ref[0:1, :] + b2_ref[0:1, :]
    y = jnp.dot(xn, w1_ref[...].T, preferred_element_type=jnp.float32)
    y = y * jax.nn.sigmoid(y)
    y = jnp.dot(y, w2_ref[...].T, preferred_element_type=jnp.float32)
    x = x_in2 + y

    wih_t = wih_ref[...].T
    whh_t = whh_ref[...].T
    bih = bih_ref[0:1, :]
    bhh = bhh_ref[0:1, :]
    q_star = jnp.zeros((1, 2 * D), jnp.float32)
    hh = jnp.zeros((1, D), jnp.float32)
    cc = jnp.zeros((1, D), jnp.float32)
    for _ in range(3):
        gates = (jnp.dot(q_star, wih_t, preferred_element_type=jnp.float32)
                 + bih
                 + jnp.dot(hh, whh_t, preferred_element_type=jnp.float32)
                 + bhh)
        ig = jax.nn.sigmoid(gates[:, 0:D])
        fg = jax.nn.sigmoid(gates[:, D:2 * D])
        gg = jnp.tanh(gates[:, 2 * D:3 * D])
        og = jax.nn.sigmoid(gates[:, 3 * D:4 * D])
        cc = fg * cc + ig * gg
        hh = og * jnp.tanh(cc)
        logits = jnp.sum(x * hh, axis=1, keepdims=True)
        lmax = jnp.max(logits, axis=0, keepdims=True)
        ex = jnp.exp(logits - lmax)
        alpha = ex / jnp.sum(ex, axis=0, keepdims=True)
        r = jnp.sum(alpha * x, axis=0, keepdims=True)
        q_star = jnp.concatenate([hh, r], axis=1)
    out_ref[...] = q_star


def _final(acc, x1, wo, bo, w1, w2, g2, b2, wih, whh, bih, bhh):
    bo8 = jnp.broadcast_to(bo[None, :], (8, D))
    g28 = jnp.broadcast_to(g2[None, :], (8, D))
    b28 = jnp.broadcast_to(b2[None, :], (8, D))
    bih8 = jnp.broadcast_to(bih[None, :], (8, 4 * D))
    bhh8 = jnp.broadcast_to(bhh[None, :], (8, 4 * D))
    return pl.pallas_call(
        _final_body,
        out_shape=jax.ShapeDtypeStruct((1, 2 * D), jnp.float32),
    )(acc, x1, wo, bo8, w1, w2, g28, b28, wih, whh, bih8, bhh8)


def kernel(node_feats, edge_feats, edge_index, Wq, Wk, Wv, We, Wo, bo, W1, W2,
           g1n, b1n, g1e, b1e, g2, b2, Wih, Whh, bih, bhh):
    src = edge_index[0].astype(jnp.int32)
    dst = edge_index[1].astype(jnp.int32)

    qt, kt, vt = _node_qkv(node_feats, Wq, Wk, Wv, g1n, b1n)

    ssum, ssq = _edge_stats(edge_feats)
    mean_e = ssum[0] / E
    var_e = ssq[0] / E - mean_e * mean_e
    s_e = g1e * lax.rsqrt(var_e + 1e-5)
    we_eff = We * s_e[None, :]
    be_eff = (b1e - mean_e * s_e) @ We.T
    pe = _proj_e(edge_feats, we_eff, be_eff)

    acc = _sc_edge(qt, kt, vt, pe, src, dst)

    return _final(acc, node_feats, Wo, bo, W1, W2, g2, b2, Wih, Whh, bih, bhh)


# speedup vs baseline: 17.0895x; 17.0895x over previous
import jax
import jax.numpy as jnp
import numpy as np
from jax import lax
from jax.experimental import pallas as pl
from jax.experimental.pallas import tpu as pltpu
from jax.experimental.pallas import tpu_sc as plsc

N = 10000
E = 320000
D = 128
H = 4
DH = 32
INV_SQRT_DH = np.float32(1.0 / np.sqrt(DH))

NC = 2
NS = 16
EDGES_PER_TILE = E // NS
CHUNK = 80
N_CHUNKS = EDGES_PER_TILE // CHUNK
NP = 10240
ROWS_PER_TILE = NP // NS
ZROWS = 128
AW = 80


def _node_qkv_body(x_ref, wq_ref, wk_ref, wv_ref, g_ref, b_ref, qt_ref, kt_ref,
                   vt_ref):
    x = x_ref[...]
    m = jnp.mean(x, axis=0, keepdims=True)
    v = jnp.mean((x - m) ** 2, axis=0, keepdims=True)
    xn = (x - m) * lax.rsqrt(v + 1e-5) * g_ref[0:1, :] + b_ref[0:1, :]
    q = jnp.dot(xn, wq_ref[...].T, preferred_element_type=jnp.float32)
    k = jnp.dot(xn, wk_ref[...].T, preferred_element_type=jnp.float32)
    k = k * INV_SQRT_DH
    w = jnp.dot(xn, wv_ref[...].T, preferred_element_type=jnp.float32)
    qt_ref[0] = q[:, 0:64]
    qt_ref[1] = q[:, 64:128]
    kt_ref[0] = k[:, 0:64]
    kt_ref[1] = k[:, 64:128]
    vt_ref[0] = w[:, 0:64]
    vt_ref[1] = w[:, 64:128]


def _node_qkv(x, wq, wk, wv, g, b):
    g8 = jnp.broadcast_to(g[None, :], (8, D))
    b8 = jnp.broadcast_to(b[None, :], (8, D))
    out = jax.ShapeDtypeStruct((NC, N, 64), jnp.float32)
    return pl.pallas_call(
        _node_qkv_body,
        out_shape=(out, out, out),
    )(x, wq, wk, wv, g8, b8)


EBLK = 2000
N_EBLK = E // EBLK


def _edge_stats_body(e_ref, s_ref, q_ref):
    i = pl.program_id(0)
    blk = e_ref[...]
    ps = jnp.sum(blk, axis=0, keepdims=True)
    pq = jnp.sum(blk * blk, axis=0, keepdims=True)
    ps8 = jnp.broadcast_to(ps, (8, D))
    pq8 = jnp.broadcast_to(pq, (8, D))

    @pl.when(i == 0)
    def _():
        s_ref[...] = ps8
        q_ref[...] = pq8

    @pl.when(i > 0)
    def _():
        s_ref[...] += ps8
        q_ref[...] += pq8


def _edge_stats(e):
    return pl.pallas_call(
        _edge_stats_body,
        grid=(N_EBLK,),
        in_specs=[pl.BlockSpec((EBLK, D), lambda i: (i, 0))],
        out_specs=(pl.BlockSpec((8, D), lambda i: (0, 0)),
                   pl.BlockSpec((8, D), lambda i: (0, 0))),
        out_shape=(jax.ShapeDtypeStruct((8, D), jnp.float32),
                   jax.ShapeDtypeStruct((8, D), jnp.float32)),
    )(e)


def _proj_e_body(e_ref, w_ref, b_ref, o_ref):
    p = jnp.dot(e_ref[...], w_ref[...].T, preferred_element_type=jnp.float32)
    p = p + b_ref[0:1, :]
    o_ref[0] = p[:, 0:64]
    o_ref[1] = p[:, 64:128]


def _proj_e(e, w_eff, b_eff):
    b8 = jnp.broadcast_to(b_eff[None, :], (8, D))
    return pl.pallas_call(
        _proj_e_body,
        grid=(N_EBLK,),
        in_specs=[pl.BlockSpec((EBLK, D), lambda i: (i, 0)),
                  pl.BlockSpec((D, D), lambda i: (0, 0)),
                  pl.BlockSpec((8, D), lambda i: (0, 0))],
        out_specs=pl.BlockSpec((NC, EBLK, 64), lambda i: (0, i, 0)),
        out_shape=jax.ShapeDtypeStruct((NC, E, 64), jnp.float32),
    )(e, w_eff, b8)


def _sc_edge_body(qt, kt, vt, pe, src, dst, out,
                  src_v, dst_v, sidx_v, didx_v,
                  krows, qrows, vrows, pe_v, contrib, zbuf, acc, sem):
    c = lax.axis_index("c")
    s = lax.axis_index("s")

    def _zero_row(i, _):
        for j in range(AW // 16):
            zbuf[i, pl.ds(16 * j, 16)] = jnp.zeros((16,), jnp.float32)
        return 0

    lax.fori_loop(0, ZROWS, _zero_row, 0)
    for k in range(ROWS_PER_TILE // ZROWS):
        pltpu.sync_copy(zbuf, acc.at[pl.ds(s * ROWS_PER_TILE + k * ZROWS,
                                           ZROWS)])
    plsc.subcore_barrier()

    def _chunk(i, _):
        base = s * EDGES_PER_TILE + i * CHUNK
        pltpu.sync_copy(src.at[pl.ds(base, CHUNK)], src_v)
        pltpu.sync_copy(dst.at[pl.ds(base, CHUNK)], dst_v)
        off = c * N
        for j in range(CHUNK // 16):
            sl = pl.ds(16 * j, 16)
            sidx_v[sl] = src_v[sl] + off
            didx_v[sl] = dst_v[sl] + off
        g1 = pltpu.async_copy(kt.at[sidx_v], krows, sem)
        g2 = pltpu.async_copy(vt.at[sidx_v], vrows, sem)
        g3 = pltpu.async_copy(qt.at[didx_v], qrows, sem)
        g4 = pltpu.async_copy(pe.at[pl.ds(c * E + base, CHUNK)], pe_v, sem)
        g1.wait()
        g2.wait()
        g3.wait()
        g4.wait()

        lanes = lax.iota(jnp.int32, 16)

        def _lane_sum(x):
            for sh in (8, 4, 2, 1):
                x = x + jnp.take_along_axis(x, lanes ^ sh, axis=0)
            return x

        def _edge(e, _):
            k0 = krows[e, pl.ds(0, 16)]
            k1 = krows[e, pl.ds(16, 16)]
            k2 = krows[e, pl.ds(32, 16)]
            k3 = krows[e, pl.ds(48, 16)]
            q0 = qrows[e, pl.ds(0, 16)]
            q1 = qrows[e, pl.ds(16, 16)]
            q2 = qrows[e, pl.ds(32, 16)]
            q3 = qrows[e, pl.ds(48, 16)]
            p0 = pe_v[e, pl.ds(0, 16)]
            p1 = pe_v[e, pl.ds(16, 16)]
            p2 = pe_v[e, pl.ds(32, 16)]
            p3 = pe_v[e, pl.ds(48, 16)]
            t0 = jnp.clip(k0 * q0, -5.0, 5.0) * p0
            t1 = jnp.clip(k1 * q1, -5.0, 5.0) * p1
            t2 = jnp.clip(k2 * q2, -5.0, 5.0) * p2
            t3 = jnp.clip(k3 * q3, -5.0, 5.0) * p3
            s0 = _lane_sum(t0 + t1)
            s1 = _lane_sum(t2 + t3)
            w0 = jnp.exp(jnp.clip(s0, -5.0, 5.0))
            w1 = jnp.exp(jnp.clip(s1, -5.0, 5.0))
            contrib[e, pl.ds(0, 16)] = vrows[e, pl.ds(0, 16)] * w0
            contrib[e, pl.ds(16, 16)] = vrows[e, pl.ds(16, 16)] * w0
            contrib[e, pl.ds(32, 16)] = vrows[e, pl.ds(32, 16)] * w1
            contrib[e, pl.ds(48, 16)] = vrows[e, pl.ds(48, 16)] * w1
            zv = jnp.where(lanes == 0, w0, jnp.where(lanes == 1, w1, 0.0))
            contrib[e, pl.ds(64, 16)] = zv
            return 0

        lax.fori_loop(0, CHUNK, _edge, 0)
        pltpu.sync_copy(contrib, acc.at[dst_v], add=True)
        return 0

    lax.fori_loop(0, N_CHUNKS, _chunk, 0)
    plsc.subcore_barrier()

    for k in range(ROWS_PER_TILE // ZROWS):
        r0 = s * ROWS_PER_TILE + k * ZROWS
        pltpu.sync_copy(acc.at[pl.ds(r0, ZROWS)], zbuf)
        pltpu.sync_copy(zbuf, out.at[c].at[pl.ds(r0, ZROWS)])


def _sc_edge(qt, kt, vt, pe, src, dst):
    qt2 = qt.reshape(NC * N, 64)
    kt2 = kt.reshape(NC * N, 64)
    vt2 = vt.reshape(NC * N, 64)
    pe2 = pe.reshape(NC * E, 64)
    mesh = plsc.VectorSubcoreMesh(core_axis_name="c", subcore_axis_name="s",
                                  num_cores=NC, num_subcores=NS)
    f = pl.kernel(
        _sc_edge_body,
        out_type=jax.ShapeDtypeStruct((NC, NP, AW), jnp.float32),
        mesh=mesh,
        compiler_params=pltpu.CompilerParams(use_tc_tiling_on_sc=False),
        scratch_types=[
            pltpu.VMEM((CHUNK,), jnp.int32),
            pltpu.VMEM((CHUNK,), jnp.int32),
            pltpu.VMEM((CHUNK,), jnp.int32),
            pltpu.VMEM((CHUNK,), jnp.int32),
            pltpu.VMEM((CHUNK, 64), jnp.float32),
            pltpu.VMEM((CHUNK, 64), jnp.float32),
            pltpu.VMEM((CHUNK, 64), jnp.float32),
            pltpu.VMEM((CHUNK, 64), jnp.float32),
            pltpu.VMEM((CHUNK, AW), jnp.float32),
            pltpu.VMEM((ZROWS, AW), jnp.float32),
            pltpu.VMEM_SHARED((NP, AW), jnp.float32),
            pltpu.SemaphoreType.DMA,
        ],
    )
    return f(qt2, kt2, vt2, pe2, src, dst)


def _final_body(acc_ref, x1_ref, wo_ref, bo_ref, w1_ref, w2_ref, g2_ref,
                b2_ref, wih_ref, whh_ref, bih_ref, bhh_ref, out_ref):
    a0 = acc_ref[0][0:N]
    a1 = acc_ref[1][0:N]
    wv = jnp.concatenate([a0[:, 0:64], a1[:, 0:64]], axis=1)
    z0 = a0[:, 64:65]
    z1 = a0[:, 65:66]
    z2 = a1[:, 64:65]
    z3 = a1[:, 65:66]
    den = jnp.concatenate([
        jnp.broadcast_to(z0, (N, DH)),
        jnp.broadcast_to(z1, (N, DH)),
        jnp.broadcast_to(z2, (N, DH)),
        jnp.broadcast_to(z3, (N, DH)),
    ], axis=1) + 1e-6
    h = wv / den
    h = jnp.dot(h, wo_ref[...].T, preferred_element_type=jnp.float32)
    h = h + bo_ref[0:1, :]
    x = x1_ref[...] + h
    x_in2 = x
    m = jnp.mean(x, axis=0, keepdims=True)
    v = jnp.mean((x - m) ** 2, axis=0, keepdims=True)
    xn = (x - m) * lax.rsqrt(v + 1e-5) * g2_ref[0:1, :] + b2_ref[0:1, :]
    y = jnp.dot(xn, w1_ref[...].T, preferred_element_type=jnp.float32)
    y = y * jax.nn.sigmoid(y)
    y = jnp.dot(y, w2_ref[...].T, preferred_element_type=jnp.float32)
    x = x_in2 + y

    wih_t = wih_ref[...].T
    whh_t = whh_ref[...].T
    bih = bih_ref[0:1, :]
    bhh = bhh_ref[0:1, :]
    q_star = jnp.zeros((1, 2 * D), jnp.float32)
    hh = jnp.zeros((1, D), jnp.float32)
    cc = jnp.zeros((1, D), jnp.float32)
    for _ in range(3):
        gates = (jnp.dot(q_star, wih_t, preferred_element_type=jnp.float32)
                 + bih
                 + jnp.dot(hh, whh_t, preferred_element_type=jnp.float32)
                 + bhh)
        ig = jax.nn.sigmoid(gates[:, 0:D])
        fg = jax.nn.sigmoid(gates[:, D:2 * D])
        gg = jnp.tanh(gates[:, 2 * D:3 * D])
        og = jax.nn.sigmoid(gates[:, 3 * D:4 * D])
        cc = fg * cc + ig * gg
        hh = og * jnp.tanh(cc)
        logits = jnp.sum(x * hh, axis=1, keepdims=True)
        lmax = jnp.max(logits, axis=0, keepdims=True)
        ex = jnp.exp(logits - lmax)
        alpha = ex / jnp.sum(ex, axis=0, keepdims=True)
        r = jnp.sum(alpha * x, axis=0, keepdims=True)
        q_star = jnp.concatenate([hh, r], axis=1)
    out_ref[...] = q_star


def _final(acc, x1, wo, bo, w1, w2, g2, b2, wih, whh, bih, bhh):
    bo8 = jnp.broadcast_to(bo[None, :], (8, D))
    g28 = jnp.broadcast_to(g2[None, :], (8, D))
    b28 = jnp.broadcast_to(b2[None, :], (8, D))
    bih8 = jnp.broadcast_to(bih[None, :], (8, 4 * D))
    bhh8 = jnp.broadcast_to(bhh[None, :], (8, 4 * D))
    return pl.pallas_call(
        _final_body,
        out_shape=jax.ShapeDtypeStruct((1, 2 * D), jnp.float32),
    )(acc, x1, wo, bo8, w1, w2, g28, b28, wih, whh, bih8, bhh8)


def kernel(node_feats, edge_feats, edge_index, Wq, Wk, Wv, We, Wo, bo, W1, W2,
           g1n, b1n, g1e, b1e, g2, b2, Wih, Whh, bih, bhh):
    src = edge_index[0].astype(jnp.int32)
    dst = edge_index[1].astype(jnp.int32)

    qt, kt, vt = _node_qkv(node_feats, Wq, Wk, Wv, g1n, b1n)

    ssum, ssq = _edge_stats(edge_feats)
    mean_e = ssum[0] / E
    var_e = ssq[0] / E - mean_e * mean_e
    s_e = g1e * lax.rsqrt(var_e + 1e-5)
    we_eff = We * s_e[None, :]
    be_eff = (b1e - mean_e * s_e) @ We.T
    pe = _proj_e(edge_feats, we_eff, be_eff)

    acc = _sc_edge(qt, kt, vt, pe, src, dst)

    return _final(acc, node_feats, Wo, bo, W1, W2, g2, b2, Wih, Whh, bih, bhh)
